# baseline (speedup 1.0000x reference)
import functools

import jax
import jax.numpy as jnp
from jax import lax
from jax.experimental import pallas as pl
from jax.experimental.pallas import tpu as pltpu
from jax.experimental.pallas import tpu_sc as plsc

N_ETHNICITIES = 1000
EMBED_DIM = 32
BATCH = 16384

_info = plsc.get_sparse_core_info()
_NC, _NS, _L = _info.num_cores, _info.num_subcores, _info.num_lanes
_NW = _NC * _NS
_B_PER_W = BATCH // _NW
_N_GROUPS = _B_PER_W // _L


@functools.partial(
    pl.kernel,
    mesh=plsc.VectorSubcoreMesh(core_axis_name="c", subcore_axis_name="s"),
    out_type=jax.ShapeDtypeStruct((EMBED_DIM, BATCH), jnp.float32),
    scratch_types=[
        pltpu.VMEM((_B_PER_W,), jnp.int32),
        pltpu.VMEM((_B_PER_W, EMBED_DIM), jnp.float32),
        pltpu.VMEM((EMBED_DIM, _B_PER_W), jnp.float32),
        pltpu.SemaphoreType.DMA,
    ],
    compiler_params=pltpu.CompilerParams(
        use_tc_tiling_on_sc=False,
        needs_layout_passes=False,
        skip_device_barrier=True,
    ),
)
def _lookup_kernel(idx_hbm, table_hbm, out_hbm, idx_v, rows_v, trows_v, sem):
    wid = lax.axis_index("s") * _NC + lax.axis_index("c")
    base = wid * _B_PER_W
    pltpu.sync_copy(idx_hbm.at[pl.ds(base, _B_PER_W)], idx_v)
    pltpu.async_copy(table_hbm.at[idx_v], rows_v, sem).wait()

    lanes = lax.iota(jnp.int32, _L)

    def body(g, _):
        bvec = g * _L + lanes
        for d in range(EMBED_DIM):
            dvec = jnp.bitwise_and(d + lanes, EMBED_DIM - 1)
            vals = plsc.load_gather(rows_v, [bvec, dvec])
            plsc.store_scatter(trows_v, [dvec, bvec], vals)
        return ()

    lax.fori_loop(0, _N_GROUPS, body, ())
    pltpu.sync_copy(trows_v, out_hbm.at[:, pl.ds(base, _B_PER_W)])


def kernel(ethnicity_idx, embedding_table):
    out_t = _lookup_kernel(ethnicity_idx.astype(jnp.int32), embedding_table)
    return out_t.T


# speedup vs baseline: 1.0232x; 1.0232x over previous
import functools

import jax
import jax.numpy as jnp
from jax import lax
from jax.experimental import pallas as pl
from jax.experimental.pallas import tpu as pltpu
from jax.experimental.pallas import tpu_sc as plsc

N_ETHNICITIES = 1000
EMBED_DIM = 32
BATCH = 16384

_info = plsc.get_sparse_core_info()
_NC, _NS, _L = _info.num_cores, _info.num_subcores, _info.num_lanes
_NW = _NC * _NS
_B_PER_W = BATCH // _NW
_N_GROUPS = _B_PER_W // _L


@functools.partial(
    pl.kernel,
    mesh=plsc.VectorSubcoreMesh(core_axis_name="c", subcore_axis_name="s"),
    out_type=jax.ShapeDtypeStruct((EMBED_DIM, BATCH), jnp.float32),
    scratch_types=[
        pltpu.VMEM((_B_PER_W,), jnp.int32),
        pltpu.VMEM((_B_PER_W, EMBED_DIM), jnp.float32),
        pltpu.VMEM((EMBED_DIM, _B_PER_W), jnp.float32),
        pltpu.SemaphoreType.DMA,
    ],
    compiler_params=pltpu.CompilerParams(
        use_tc_tiling_on_sc=False,
        needs_layout_passes=False,
        skip_device_barrier=True,
    ),
)
def _lookup_kernel(idx_hbm, table_hbm, out_hbm, idx_v, rows_v, trows_v, sem):
    wid = lax.axis_index("s") * _NC + lax.axis_index("c")
    base = wid * _B_PER_W
    pltpu.sync_copy(idx_hbm.at[pl.ds(base, _B_PER_W)], idx_v)
    pltpu.async_copy(table_hbm.at[idx_v], rows_v, sem).wait()

    lanes = lax.iota(jnp.int32, _L)

    def body(g, _):
        bvec = g * _L + lanes

        def dbody(d, _):
            dvec = jnp.bitwise_and(d + lanes, EMBED_DIM - 1)
            vals = plsc.load_gather(rows_v, [bvec, dvec])
            plsc.store_scatter(trows_v, [dvec, bvec], vals)
            return ()

        lax.fori_loop(0, EMBED_DIM, dbody, ())
        return ()

    lax.fori_loop(0, _N_GROUPS, body, ())
    pltpu.sync_copy(trows_v, out_hbm.at[:, pl.ds(base, _B_PER_W)])


def kernel(ethnicity_idx, embedding_table):
    out_t = _lookup_kernel(ethnicity_idx.astype(jnp.int32), embedding_table)
    return out_t.T


# speedup vs baseline: 1.0393x; 1.0157x over previous
import functools

import jax
import jax.numpy as jnp
from jax import lax
from jax.experimental import pallas as pl
from jax.experimental.pallas import tpu as pltpu
from jax.experimental.pallas import tpu_sc as plsc

N_ETHNICITIES = 1000
EMBED_DIM = 32
BATCH = 16384

_info = plsc.get_sparse_core_info()
_NC, _NS, _L = _info.num_cores, _info.num_subcores, _info.num_lanes
_NW = _NC * _NS
_B_PER_W = BATCH // _NW
_N_GROUPS = _B_PER_W // _L


@functools.partial(
    pl.kernel,
    mesh=plsc.VectorSubcoreMesh(core_axis_name="c", subcore_axis_name="s"),
    out_type=jax.ShapeDtypeStruct((EMBED_DIM, BATCH), jnp.float32),
    scratch_types=[
        pltpu.VMEM((_B_PER_W,), jnp.int32),
        pltpu.VMEM((_B_PER_W, EMBED_DIM), jnp.float32),
        pltpu.VMEM((EMBED_DIM, _B_PER_W), jnp.float32),
        pltpu.SemaphoreType.DMA,
    ],
    compiler_params=pltpu.CompilerParams(
        use_tc_tiling_on_sc=False,
        needs_layout_passes=False,
        skip_device_barrier=True,
    ),
)
def _lookup_kernel(idx_hbm, table_hbm, out_hbm, idx_v, rows_v, trows_v, sem):
    wid = lax.axis_index("s") * _NC + lax.axis_index("c")
    base = wid * _B_PER_W
    pltpu.sync_copy(idx_hbm.at[pl.ds(base, _B_PER_W)], idx_v)
    pltpu.async_copy(table_hbm.at[idx_v], rows_v, sem).wait()

    lanes = lax.iota(jnp.int32, _L)

    def body(g, _):
        bvec = g * _L + lanes

        def dbody(d8, _):
            for u in range(8):
                dvec = jnp.bitwise_and(d8 * 8 + u + lanes, EMBED_DIM - 1)
                vals = plsc.load_gather(rows_v, [bvec, dvec])
                plsc.store_scatter(trows_v, [dvec, bvec], vals)
            return ()

        lax.fori_loop(0, EMBED_DIM // 8, dbody, ())
        return ()

    lax.fori_loop(0, _N_GROUPS, body, ())
    pltpu.sync_copy(trows_v, out_hbm.at[:, pl.ds(base, _B_PER_W)])


def kernel(ethnicity_idx, embedding_table):
    out_t = _lookup_kernel(ethnicity_idx.astype(jnp.int32), embedding_table)
    return out_t.T
